# baseline (speedup 1.0000x reference)
import math

import jax
import jax.numpy as jnp
from jax import lax
from jax.experimental import pallas as pl
from jax.experimental.pallas import tpu as pltpu

_GX, _GY, _GZ = 256, 256, 256
_SIGMA = 0.8
_TIME_CONST = 1000.0 / math.log(_SIGMA)

_NBLK = 16
_ROWS = (_GX * _GY) // _NBLK


def _body(denom_ref, x_ref, w_ref, o_ref, maxval, maxidx):
    i = pl.program_id(0)

    scores = jnp.dot(w_ref[...], x_ref[...], preferred_element_type=jnp.float32)
    s2 = scores.reshape(_ROWS // 128, 128)

    bm = jnp.max(s2)
    ii = lax.broadcasted_iota(jnp.int32, s2.shape, 0) * 128 + lax.broadcasted_iota(
        jnp.int32, s2.shape, 1
    )
    bidx = jnp.min(jnp.where(s2 == bm, ii, jnp.int32(2**30)))

    prev = jnp.where(i == 0, -jnp.inf, maxval[0])
    better = bm > prev
    maxval[0] = jnp.where(better, bm, prev)
    maxidx[0] = jnp.where(better, i * _ROWS + bidx, jnp.where(i == 0, 0, maxidx[0]))

    @pl.when(i == _NBLK - 1)
    def _():
        wflat = maxidx[0]
        wi = (wflat // _GY).astype(jnp.float32)
        wj = (wflat % _GY).astype(jnp.float32)
        den = denom_ref[0]
        gi = lax.broadcasted_iota(jnp.int32, (_GX, _GY), 0).astype(jnp.float32)
        gj = lax.broadcasted_iota(jnp.int32, (_GX, _GY), 1).astype(jnp.float32)
        o_ref[...] = jnp.exp(-((gi - wi) ** 2 / den)) * jnp.exp(-((gj - wj) ** 2 / den))


def kernel(x, t, W):
    decay = _SIGMA * jnp.exp(-t / _TIME_CONST)
    denom = (2.0 * decay * decay).astype(jnp.float32).reshape(1)
    W2 = W.reshape(_GX * _GY, _GZ)
    x2 = x.reshape(_GZ, 1)

    out = pl.pallas_call(
        _body,
        grid=(_NBLK,),
        in_specs=[
            pl.BlockSpec(memory_space=pltpu.SMEM),
            pl.BlockSpec((_GZ, 1), lambda i: (0, 0)),
            pl.BlockSpec((_ROWS, _GZ), lambda i: (i, 0)),
        ],
        out_specs=pl.BlockSpec((_GX, _GY), lambda i: (0, 0)),
        out_shape=jax.ShapeDtypeStruct((_GX, _GY), jnp.float32),
        scratch_shapes=[
            pltpu.SMEM((1,), jnp.float32),
            pltpu.SMEM((1,), jnp.int32),
        ],
    )(denom, x2, W2)
    return out


# speedup vs baseline: 1.2739x; 1.2739x over previous
import math

import jax
import jax.numpy as jnp
from jax import lax
from jax.experimental import pallas as pl
from jax.experimental.pallas import tpu as pltpu

_GX, _GY, _GZ = 256, 256, 256
_SIGMA = 0.8
_TIME_CONST = 1000.0 / math.log(_SIGMA)

_NBLK = 16
_ROWS = (_GX * _GY) // _NBLK


def _body(denom_ref, x_ref, w_ref, o_ref, maxval, maxidx):
    i = pl.program_id(0)

    wv = w_ref[...].reshape(_ROWS, _GZ)
    scores = jnp.dot(wv, x_ref[...], preferred_element_type=jnp.float32)

    bm = jnp.max(scores)
    better = jnp.logical_or(i == 0, bm > maxval[0])

    @pl.when(better)
    def _():
        ii = lax.broadcasted_iota(jnp.int32, scores.shape, 0)
        bidx = jnp.min(jnp.where(scores == bm, ii, jnp.int32(2**30)))
        maxval[0] = bm
        maxidx[0] = i * _ROWS + bidx

    @pl.when(i == _NBLK - 1)
    def _():
        wflat = maxidx[0]
        wi = (wflat // _GY).astype(jnp.float32)
        wj = (wflat % _GY).astype(jnp.float32)
        den = denom_ref[0]
        gi = lax.broadcasted_iota(jnp.int32, (_GX, _GY), 0).astype(jnp.float32)
        gj = lax.broadcasted_iota(jnp.int32, (_GX, _GY), 1).astype(jnp.float32)
        o_ref[...] = jnp.exp(-((gi - wi) ** 2 / den)) * jnp.exp(-((gj - wj) ** 2 / den))


def kernel(x, t, W):
    decay = _SIGMA * jnp.exp(-t / _TIME_CONST)
    denom = (2.0 * decay * decay).astype(jnp.float32).reshape(1)
    x2 = x.reshape(_GZ, 1)

    out = pl.pallas_call(
        _body,
        grid=(_NBLK,),
        in_specs=[
            pl.BlockSpec(memory_space=pltpu.SMEM),
            pl.BlockSpec((_GZ, 1), lambda i: (0, 0)),
            pl.BlockSpec((_GX // _NBLK, _GY, _GZ), lambda i: (i, 0, 0)),
        ],
        out_specs=pl.BlockSpec((_GX, _GY), lambda i: (0, 0)),
        out_shape=jax.ShapeDtypeStruct((_GX, _GY), jnp.float32),
        scratch_shapes=[
            pltpu.SMEM((1,), jnp.float32),
            pltpu.SMEM((1,), jnp.int32),
        ],
    )(denom, x2, W)
    return out
